# Initial kernel scaffold; baseline (speedup 1.0000x reference)
import jax
import jax.numpy as jnp
from jax.experimental import pallas as pl


def kernel(x, xyz_nb, xyz_id, dij, params):
    raise NotImplementedError("write your pallas kernel here")


# speedup vs baseline: 1.7882x; 1.7882x over previous
import functools
import math

import jax
import jax.numpy as jnp
from jax import lax
from jax.experimental import pallas as pl
from jax.experimental.pallas import tpu as pltpu
from jax.experimental.pallas import tpu_sc as plsc

N = 10000
K = 32
D = 128
H = 64
NK = N * K
THR = 13.0
EPS = 1e-5
ALPHA = 0.7
LAMBDA = 1.5

NC = 2
NS = 16
NW = NC * NS
CHUNK = 320
NP = NW * CHUNK
G = 4
NG = CHUNK // G
NPK = NP * K


def _leaky(v):
    return jnp.where(v >= 0, v, 0.2 * v)


def _geom_body(dij_ref, xyz_ref, wc1_0, bc1_0, wc2_0, bc2_0,
               wc1_1, bc1_1, wc2_1, bc2_1, wc1_2, bc1_2, wc2_2, bc2_2,
               w0_ref, w1_ref, w2_ref, win_ref):
    dij = dij_ref[...]
    d2 = jnp.sum(dij * dij, axis=1, keepdims=True)
    win = jnp.exp(-d2 / (2.0 * THR * THR)) * (d2 < THR * THR).astype(jnp.float32)
    win_ref[...] = win
    xyz = xyz_ref[...]
    for (wc1, bc1, wc2, bc2, out) in (
        (wc1_0, bc1_0, wc2_0, bc2_0, w0_ref),
        (wc1_1, bc1_1, wc2_1, bc2_1, w1_ref),
        (wc1_2, bc1_2, wc2_2, bc2_2, w2_ref),
    ):
        t = jnp.maximum(jnp.dot(xyz, wc1[...]) + bc1[...], 0.0)
        p3 = jnp.dot(t, wc2[...]) + bc2[...]
        out[...] = win * p3


def _geom_call(dij_flat, xyz_flat, convs):
    BK = 8000
    grid = (NK // BK,)
    row = lambda w: pl.BlockSpec((BK, w), lambda i: (i, 0))
    full = lambda a: pl.BlockSpec(a.shape, lambda i: (0, 0))
    args = [dij_flat, xyz_flat]
    specs = [row(3), row(3)]
    for (wc1, bc1, wc2, bc2) in convs:
        args += [wc1, bc1, wc2, bc2]
        specs += [full(wc1), full(bc1), full(wc2), full(bc2)]
    out_shapes = [jax.ShapeDtypeStruct((NK, 3), jnp.float32)] * 3 + [
        jax.ShapeDtypeStruct((NK, 1), jnp.float32)]
    out_specs = [row(3)] * 3 + [row(1)]
    return pl.pallas_call(
        _geom_body, grid=grid, in_specs=specs, out_specs=out_specs,
        out_shape=out_shapes)(*args)


def _wsum_body(win_ref, out_ref):
    out_ref[...] = jnp.sum(win_ref[...], axis=1, keepdims=True)


def _wsum_call(window2d):
    BN = 1000
    return pl.pallas_call(
        _wsum_body, grid=(N // BN,),
        in_specs=[pl.BlockSpec((BN, K), lambda i: (i, 0))],
        out_specs=pl.BlockSpec((BN, 1), lambda i: (i, 0)),
        out_shape=jax.ShapeDtypeStruct((N, 1), jnp.float32))(window2d)


def _linear_body(x_ref, w_ref, b_ref, o_ref):
    o_ref[...] = jnp.dot(x_ref[...], w_ref[...]) + b_ref[...]


def _linear_call(x, w, b):
    BN = 1000
    din, dout = w.shape
    return pl.pallas_call(
        _linear_body, grid=(N // BN,),
        in_specs=[pl.BlockSpec((BN, din), lambda i: (i, 0)),
                  pl.BlockSpec((din, dout), lambda i: (0, 0)),
                  pl.BlockSpec((1, dout), lambda i: (0, 0))],
        out_specs=pl.BlockSpec((BN, dout), lambda i: (i, 0)),
        out_shape=jax.ShapeDtypeStruct((N, dout), jnp.float32))(x, w, b)


def _mlp_body(h_ref, w1_ref, b1_ref, w2_ref, b2_ref, q_ref):
    t = _leaky(jnp.dot(h_ref[...], w1_ref[...]) + b1_ref[...])
    q = _leaky(jnp.dot(t, w2_ref[...]) + b2_ref[...])
    q_ref[...] = jnp.concatenate(
        [q, jnp.zeros_like(q)], axis=1)


def _mlp_call(h, w1, b1, w2, b2):
    BN = 1000
    din = h.shape[1]
    return pl.pallas_call(
        _mlp_body, grid=(N // BN,),
        in_specs=[pl.BlockSpec((BN, din), lambda i: (i, 0)),
                  pl.BlockSpec((din, H), lambda i: (0, 0)),
                  pl.BlockSpec((1, H), lambda i: (0, 0)),
                  pl.BlockSpec((H, H), lambda i: (0, 0)),
                  pl.BlockSpec((1, H), lambda i: (0, 0))],
        out_specs=pl.BlockSpec((BN, 2 * H), lambda i: (i, 0)),
        out_shape=jax.ShapeDtypeStruct((N, 2 * H), jnp.float32))(
            h, w1, b1, w2, b2)


def _post_body(theta, final, s_ref, ws_ref, res_ref, wm_ref, bm_ref, g_ref,
               b_ref, l1_ref, lb1_ref, l2_ref, lb2_ref, ch_ref,
               wo1_ref, bo1_ref, wo2_ref, bo2_ref, o_ref):
    F = jnp.dot(s_ref[...], wm_ref[...]) + ws_ref[...] * bm_ref[...]
    F1 = F * (g_ref[...] / math.sqrt(1.0 + EPS)) + b_ref[...]
    lin = jnp.maximum(jnp.dot(F1, l1_ref[...]) + lb1_ref[...], 0.0)
    lin = jnp.dot(lin, l2_ref[...]) + lb2_ref[...]
    sup = (1.0 - ALPHA) * lin + ALPHA * res_ref[...]
    out = theta * jnp.dot(sup, ch_ref[...]) + (1.0 - theta) * sup + lin
    h = jnp.maximum(out, 0.0)
    if final:
        o = _leaky(jnp.dot(h, wo1_ref[...]) + bo1_ref[...])
        o = _leaky(o * wo2_ref[...] + bo2_ref[...])
        o_ref[...] = jax.nn.sigmoid(o)
    else:
        o_ref[...] = h


def _post_call(theta, final, s, wsum, res, wm, bm, g, b, l1, lb1, l2, lb2,
               ch, wo1, bo1, wo2, bo2):
    BN = 1000
    full = lambda a: pl.BlockSpec(a.shape, lambda i: (0, 0))
    dout = 1 if final else H
    return pl.pallas_call(
        functools.partial(_post_body, theta, final), grid=(N // BN,),
        in_specs=[pl.BlockSpec((BN, 3 * H), lambda i: (i, 0)),
                  pl.BlockSpec((BN, 1), lambda i: (i, 0)),
                  pl.BlockSpec((BN, H), lambda i: (i, 0)),
                  full(wm), full(bm), full(g), full(b), full(l1), full(lb1),
                  full(l2), full(lb2), full(ch), full(wo1), full(bo1),
                  full(wo2), full(bo2)],
        out_specs=pl.BlockSpec((BN, dout), lambda i: (i, 0)),
        out_shape=jax.ShapeDtypeStruct((N, dout), jnp.float32))(
            s, wsum, res, wm, bm, g, b, l1, lb1, l2, lb2, ch, wo1, bo1,
            wo2, bo2)


def _sc_gather_body(q_hbm, idx_hbm, wgt_hbm, out_hbm, idx_g, w_g, rows, out_g,
                    sem):
    wid = lax.axis_index("s") * NC + lax.axis_index("c")
    base_k = wid * (CHUNK * K)
    base_w = wid * (CHUNK * K * 3)
    base_o = wid * (CHUNK * 3 * H)

    def group(g, carry):
        pltpu.sync_copy(idx_hbm.at[pl.ds(base_k + g * (G * K), G * K)], idx_g)
        pltpu.sync_copy(wgt_hbm.at[pl.ds(base_w + g * (G * K * 3), G * K * 3)],
                        w_g)
        pltpu.async_copy(q_hbm.at[idx_g], rows, sem).wait()
        for i in range(G):
            wv = [w_g[pl.ds(i * (K * 3) + v * 16, 16)] for v in range(K * 3 // 16)]
            acc = [[jnp.zeros((16,), jnp.float32) for _ in range(H // 16)]
                   for _ in range(3)]
            for k in range(K):
                r = [rows[i * K + k, pl.ds(hc * 16, 16)] for hc in range(H // 16)]
                for c in range(3):
                    p = 3 * k + c
                    lane = jnp.full((16,), p % 16, jnp.int32)
                    w_sp = wv[p // 16].at[lane].get(mode='promise_in_bounds')
                    for hc in range(H // 16):
                        acc[c][hc] = acc[c][hc] + w_sp * r[hc]
            for c in range(3):
                for hc in range(H // 16):
                    out_g[pl.ds(i * (3 * H) + c * H + hc * 16, 16)] = acc[c][hc]
        pltpu.sync_copy(out_g,
                        out_hbm.at[pl.ds(base_o + g * (G * 3 * H), G * 3 * H)])
        return carry

    lax.fori_loop(0, NG, group, 0)


@functools.cache
def _get_sc_gather():
    return functools.partial(
        pl.kernel,
        out_type=jax.ShapeDtypeStruct((NP * 3 * H,), jnp.float32),
        mesh=plsc.VectorSubcoreMesh(core_axis_name="c", subcore_axis_name="s",
                                    num_cores=NC, num_subcores=NS),
        scratch_types=[
            pltpu.VMEM((G * K,), jnp.int32),
            pltpu.VMEM((G * K * 3,), jnp.float32),
            pltpu.VMEM((G * K, 2 * H), jnp.float32),
            pltpu.VMEM((G * 3 * H,), jnp.float32),
            pltpu.SemaphoreType.DMA,
        ],
    )(_sc_gather_body)


def kernel(x, xyz_nb, xyz_id, dij, params):
    dij_flat = dij.reshape(NK, 3)
    xyz_flat = xyz_nb.reshape(NK, 3)
    convs = []
    for j in range(3):
        wc1, bc1, wc2, bc2 = params['conv%d' % j]
        convs.append((wc1, bc1.reshape(1, 16), wc2[:, :3],
                      bc2[:3].reshape(1, 3)))
    w0, w1, w2, window = _geom_call(dij_flat, xyz_flat, convs)
    wgt = [w0, w1, w2]
    wsum = _wsum_call(window.reshape(N, K))

    wfc, bfc = params['fc']
    residual = _linear_call(x, wfc, bfc.reshape(1, H))

    idx_pad = jnp.pad(xyz_id.astype(jnp.int32).reshape(-1), (0, NPK - NK))

    h = x
    o = None
    for j in range(3):
        w1_, b1_, w2_, b2_ = params['net_in%d' % j]
        q = _mlp_call(h, w1_, b1_.reshape(1, H), w2_, b2_.reshape(1, H))
        wgt_flat = jnp.pad(wgt[j].reshape(-1), (0, (NPK - NK) * 3))
        s = _get_sc_gather()(q, idx_pad, wgt_flat).reshape(NP, 3 * H)[:N]
        wm, bm = params['mlp']
        g, b = params['bn%d' % j]
        l1, lb1, l2, lb2 = params['lin%d' % j]
        wo1, bo1, wo2, bo2 = params['net_out']
        theta = min(1.0, math.log(LAMBDA / (j + 1) + 1.0))
        final = j == 2
        res = _post_call(theta, final, s, wsum, residual, wm,
                         bm.reshape(1, H), g.reshape(1, H), b.reshape(1, H),
                         l1, lb1.reshape(1, H), l2, lb2.reshape(1, H),
                         params['chen%d' % j], wo1, bo1.reshape(1, 1), wo2,
                         bo2.reshape(1, 1))
        if final:
            o = res
        else:
            h = res
    return o.reshape(N)


# speedup vs baseline: 2.0869x; 1.1670x over previous
import functools
import math

import jax
import jax.numpy as jnp
from jax import lax
from jax.experimental import pallas as pl
from jax.experimental.pallas import tpu as pltpu
from jax.experimental.pallas import tpu_sc as plsc

N = 10000
K = 32
D = 128
H = 64
NK = N * K
THR = 13.0
EPS = 1e-5
ALPHA = 0.7
LAMBDA = 1.5

NC = 2
NS = 16
NW = NC * NS
CHUNK = 320
NP = NW * CHUNK
G = 2
NG = CHUNK // G
NPK = NP * K


def _leaky(v):
    return jnp.where(v >= 0, v, 0.2 * v)


def _geom_body(dij_ref, xyz_ref, wc1_0, bc1_0, wc2_0, bc2_0,
               wc1_1, bc1_1, wc2_1, bc2_1, wc1_2, bc1_2, wc2_2, bc2_2,
               w0_ref, w1_ref, w2_ref, win_ref):
    dij = dij_ref[...]
    d2 = jnp.sum(dij * dij, axis=1, keepdims=True)
    win = jnp.exp(-d2 / (2.0 * THR * THR)) * (d2 < THR * THR).astype(jnp.float32)
    win_ref[...] = win
    xyz = xyz_ref[...]
    for (wc1, bc1, wc2, bc2, out) in (
        (wc1_0, bc1_0, wc2_0, bc2_0, w0_ref),
        (wc1_1, bc1_1, wc2_1, bc2_1, w1_ref),
        (wc1_2, bc1_2, wc2_2, bc2_2, w2_ref),
    ):
        t = jnp.maximum(jnp.dot(xyz, wc1[...]) + bc1[...], 0.0)
        p3 = jnp.dot(t, wc2[...]) + bc2[...]
        out[...] = win * p3


def _geom_call(dij_flat, xyz_flat, convs):
    BK = 8000
    grid = (NK // BK,)
    row = lambda w: pl.BlockSpec((BK, w), lambda i: (i, 0))
    full = lambda a: pl.BlockSpec(a.shape, lambda i: (0, 0))
    args = [dij_flat, xyz_flat]
    specs = [row(3), row(3)]
    for (wc1, bc1, wc2, bc2) in convs:
        args += [wc1, bc1, wc2, bc2]
        specs += [full(wc1), full(bc1), full(wc2), full(bc2)]
    out_shapes = [jax.ShapeDtypeStruct((NK, 3), jnp.float32)] * 3 + [
        jax.ShapeDtypeStruct((NK, 1), jnp.float32)]
    out_specs = [row(3)] * 3 + [row(1)]
    return pl.pallas_call(
        _geom_body, grid=grid, in_specs=specs, out_specs=out_specs,
        out_shape=out_shapes)(*args)


def _wsum_body(win_ref, out_ref):
    out_ref[...] = jnp.sum(win_ref[...], axis=1, keepdims=True)


def _wsum_call(window2d):
    BN = 1000
    return pl.pallas_call(
        _wsum_body, grid=(N // BN,),
        in_specs=[pl.BlockSpec((BN, K), lambda i: (i, 0))],
        out_specs=pl.BlockSpec((BN, 1), lambda i: (i, 0)),
        out_shape=jax.ShapeDtypeStruct((N, 1), jnp.float32))(window2d)


def _linear_body(x_ref, w_ref, b_ref, o_ref):
    o_ref[...] = jnp.dot(x_ref[...], w_ref[...]) + b_ref[...]


def _linear_call(x, w, b):
    BN = 1000
    din, dout = w.shape
    return pl.pallas_call(
        _linear_body, grid=(N // BN,),
        in_specs=[pl.BlockSpec((BN, din), lambda i: (i, 0)),
                  pl.BlockSpec((din, dout), lambda i: (0, 0)),
                  pl.BlockSpec((1, dout), lambda i: (0, 0))],
        out_specs=pl.BlockSpec((BN, dout), lambda i: (i, 0)),
        out_shape=jax.ShapeDtypeStruct((N, dout), jnp.float32))(x, w, b)


def _mlp_body(h_ref, w1_ref, b1_ref, w2_ref, b2_ref, q_ref):
    t = _leaky(jnp.dot(h_ref[...], w1_ref[...]) + b1_ref[...])
    q = _leaky(jnp.dot(t, w2_ref[...]) + b2_ref[...])
    q_ref[...] = jnp.concatenate(
        [q, jnp.zeros_like(q)], axis=1)


def _mlp_call(h, w1, b1, w2, b2):
    BN = 1000
    din = h.shape[1]
    return pl.pallas_call(
        _mlp_body, grid=(N // BN,),
        in_specs=[pl.BlockSpec((BN, din), lambda i: (i, 0)),
                  pl.BlockSpec((din, H), lambda i: (0, 0)),
                  pl.BlockSpec((1, H), lambda i: (0, 0)),
                  pl.BlockSpec((H, H), lambda i: (0, 0)),
                  pl.BlockSpec((1, H), lambda i: (0, 0))],
        out_specs=pl.BlockSpec((BN, 2 * H), lambda i: (i, 0)),
        out_shape=jax.ShapeDtypeStruct((N, 2 * H), jnp.float32))(
            h, w1, b1, w2, b2)


def _post_body(theta, final, s_ref, ws_ref, res_ref, wm_ref, bm_ref, g_ref,
               b_ref, l1_ref, lb1_ref, l2_ref, lb2_ref, ch_ref,
               wo1_ref, bo1_ref, wo2_ref, bo2_ref, o_ref):
    F = jnp.dot(s_ref[...], wm_ref[...]) + ws_ref[...] * bm_ref[...]
    F1 = F * (g_ref[...] / math.sqrt(1.0 + EPS)) + b_ref[...]
    lin = jnp.maximum(jnp.dot(F1, l1_ref[...]) + lb1_ref[...], 0.0)
    lin = jnp.dot(lin, l2_ref[...]) + lb2_ref[...]
    sup = (1.0 - ALPHA) * lin + ALPHA * res_ref[...]
    out = theta * jnp.dot(sup, ch_ref[...]) + (1.0 - theta) * sup + lin
    h = jnp.maximum(out, 0.0)
    if final:
        o = _leaky(jnp.dot(h, wo1_ref[...]) + bo1_ref[...])
        o = _leaky(o * wo2_ref[...] + bo2_ref[...])
        o_ref[...] = jax.nn.sigmoid(o)
    else:
        o_ref[...] = h


def _post_call(theta, final, s, wsum, res, wm, bm, g, b, l1, lb1, l2, lb2,
               ch, wo1, bo1, wo2, bo2):
    BN = 1000
    full = lambda a: pl.BlockSpec(a.shape, lambda i: (0, 0))
    dout = 1 if final else H
    return pl.pallas_call(
        functools.partial(_post_body, theta, final), grid=(N // BN,),
        in_specs=[pl.BlockSpec((BN, 3 * H), lambda i: (i, 0)),
                  pl.BlockSpec((BN, 1), lambda i: (i, 0)),
                  pl.BlockSpec((BN, H), lambda i: (i, 0)),
                  full(wm), full(bm), full(g), full(b), full(l1), full(lb1),
                  full(l2), full(lb2), full(ch), full(wo1), full(bo1),
                  full(wo2), full(bo2)],
        out_specs=pl.BlockSpec((BN, dout), lambda i: (i, 0)),
        out_shape=jax.ShapeDtypeStruct((N, dout), jnp.float32))(
            s, wsum, res, wm, bm, g, b, l1, lb1, l2, lb2, ch, wo1, bo1,
            wo2, bo2)


def _sc_gather_body(q_hbm, idx_hbm, wgt_hbm, out_hbm, idx_all, w_all, rows2,
                    out_half, sem0, sem1):
    wid = lax.axis_index("s") * NC + lax.axis_index("c")
    base_k = wid * (CHUNK * K)
    base_w = wid * (CHUNK * K * 3)
    base_o = wid * (CHUNK * 3 * H)
    HG = NG // 2
    GW = G * 3 * H

    pltpu.sync_copy(idx_hbm.at[pl.ds(base_k, CHUNK * K)], idx_all)
    pltpu.sync_copy(wgt_hbm.at[pl.ds(base_w, CHUNK * K * 3)], w_all)

    def gather(g, buf, sem):
        return pltpu.make_async_copy(
            q_hbm.at[idx_all.at[pl.ds(g * (G * K), G * K)]], buf, sem)

    gather(0, rows2.at[0], sem0).start()
    gather(1, rows2.at[1], sem1).start()

    def compute(g, buf):
        off = (g % HG) * GW
        for i in range(G):
            node = g * G + i
            wbase = node * (K * 3)
            wv = [w_all[pl.ds(wbase + v * 16, 16)] for v in range(K * 3 // 16)]
            acc = [[jnp.zeros((16,), jnp.float32) for _ in range(H // 16)]
                   for _ in range(3)]
            for k in range(K):
                r = [buf[i * K + k, pl.ds(hc * 16, 16)]
                     for hc in range(H // 16)]
                for c in range(3):
                    p = 3 * k + c
                    lane = jnp.full((16,), p % 16, jnp.int32)
                    w_sp = wv[p // 16].at[lane].get(mode='promise_in_bounds')
                    for hc in range(H // 16):
                        acc[c][hc] = acc[c][hc] + w_sp * r[hc]
            for c in range(3):
                for hc in range(H // 16):
                    out_half[pl.ds(off + i * (3 * H) + c * H + hc * 16, 16)] = \
                        acc[c][hc]

    def body2(t, carry):
        g0 = 2 * t
        g1 = g0 + 1
        gather(g0, rows2.at[0], sem0).wait()
        compute(g0, rows2.at[0])

        @pl.when(t + 1 < NG // 2)
        def _():
            gather(g0 + 2, rows2.at[0], sem0).start()

        gather(g1, rows2.at[1], sem1).wait()
        compute(g1, rows2.at[1])

        @pl.when(t + 1 < NG // 2)
        def _():
            gather(g1 + 2, rows2.at[1], sem1).start()

        @pl.when(g1 == HG - 1)
        def _():
            pltpu.sync_copy(out_half, out_hbm.at[pl.ds(base_o, HG * GW)])

        @pl.when(g1 == NG - 1)
        def _():
            pltpu.sync_copy(out_half,
                            out_hbm.at[pl.ds(base_o + HG * GW, HG * GW)])

        return carry

    lax.fori_loop(0, NG // 2, body2, 0)


@functools.cache
def _get_sc_gather():
    return functools.partial(
        pl.kernel,
        out_type=jax.ShapeDtypeStruct((NP * 3 * H,), jnp.float32),
        mesh=plsc.VectorSubcoreMesh(core_axis_name="c", subcore_axis_name="s",
                                    num_cores=NC, num_subcores=NS),
        scratch_types=[
            pltpu.VMEM((CHUNK * K,), jnp.int32),
            pltpu.VMEM((CHUNK * K * 3,), jnp.float32),
            pltpu.VMEM((2, G * K, 2 * H), jnp.float32),
            pltpu.VMEM(((NG // 2) * G * 3 * H,), jnp.float32),
            pltpu.SemaphoreType.DMA,
            pltpu.SemaphoreType.DMA,
        ],
    )(_sc_gather_body)


def kernel(x, xyz_nb, xyz_id, dij, params):
    dij_flat = dij.reshape(NK, 3)
    xyz_flat = xyz_nb.reshape(NK, 3)
    convs = []
    for j in range(3):
        wc1, bc1, wc2, bc2 = params['conv%d' % j]
        convs.append((wc1, bc1.reshape(1, 16), wc2[:, :3],
                      bc2[:3].reshape(1, 3)))
    w0, w1, w2, window = _geom_call(dij_flat, xyz_flat, convs)
    wgt = [w0, w1, w2]
    wsum = _wsum_call(window.reshape(N, K))

    wfc, bfc = params['fc']
    residual = _linear_call(x, wfc, bfc.reshape(1, H))

    idx_pad = jnp.pad(xyz_id.astype(jnp.int32).reshape(-1), (0, NPK - NK))

    h = x
    o = None
    for j in range(3):
        w1_, b1_, w2_, b2_ = params['net_in%d' % j]
        q = _mlp_call(h, w1_, b1_.reshape(1, H), w2_, b2_.reshape(1, H))
        wgt_flat = jnp.pad(wgt[j].reshape(-1), (0, (NPK - NK) * 3))
        s = _get_sc_gather()(q, idx_pad, wgt_flat).reshape(NP, 3 * H)[:N]
        wm, bm = params['mlp']
        g, b = params['bn%d' % j]
        l1, lb1, l2, lb2 = params['lin%d' % j]
        wo1, bo1, wo2, bo2 = params['net_out']
        theta = min(1.0, math.log(LAMBDA / (j + 1) + 1.0))
        final = j == 2
        res = _post_call(theta, final, s, wsum, residual, wm,
                         bm.reshape(1, H), g.reshape(1, H), b.reshape(1, H),
                         l1, lb1.reshape(1, H), l2, lb2.reshape(1, H),
                         params['chen%d' % j], wo1, bo1.reshape(1, 1), wo2,
                         bo2.reshape(1, 1))
        if final:
            o = res
        else:
            h = res
    return o.reshape(N)


# speedup vs baseline: 2.0958x; 1.0043x over previous
import functools
import math

import jax
import jax.numpy as jnp
from jax import lax
from jax.experimental import pallas as pl
from jax.experimental.pallas import tpu as pltpu
from jax.experimental.pallas import tpu_sc as plsc

N = 10000
K = 32
D = 128
H = 64
NK = N * K
THR = 13.0
EPS = 1e-5
ALPHA = 0.7
LAMBDA = 1.5

NC = 2
NS = 16
NW = NC * NS
CHUNK = 320
NP = NW * CHUNK
G = 1
NG = CHUNK // G
RING = 4
NPK = NP * K


def _leaky(v):
    return jnp.where(v >= 0, v, 0.2 * v)


def _geom_body(dij_ref, xyz_ref, wc1_0, bc1_0, wc2_0, bc2_0,
               wc1_1, bc1_1, wc2_1, bc2_1, wc1_2, bc1_2, wc2_2, bc2_2,
               w0_ref, w1_ref, w2_ref, win_ref):
    dij = dij_ref[...]
    d2 = jnp.sum(dij * dij, axis=1, keepdims=True)
    win = jnp.exp(-d2 / (2.0 * THR * THR)) * (d2 < THR * THR).astype(jnp.float32)
    win_ref[...] = win
    xyz = xyz_ref[...]
    for (wc1, bc1, wc2, bc2, out) in (
        (wc1_0, bc1_0, wc2_0, bc2_0, w0_ref),
        (wc1_1, bc1_1, wc2_1, bc2_1, w1_ref),
        (wc1_2, bc1_2, wc2_2, bc2_2, w2_ref),
    ):
        t = jnp.maximum(jnp.dot(xyz, wc1[...]) + bc1[...], 0.0)
        p3 = jnp.dot(t, wc2[...]) + bc2[...]
        out[...] = win * p3


def _geom_call(dij_flat, xyz_flat, convs):
    BK = 8000
    grid = (NK // BK,)
    row = lambda w: pl.BlockSpec((BK, w), lambda i: (i, 0))
    full = lambda a: pl.BlockSpec(a.shape, lambda i: (0, 0))
    args = [dij_flat, xyz_flat]
    specs = [row(3), row(3)]
    for (wc1, bc1, wc2, bc2) in convs:
        args += [wc1, bc1, wc2, bc2]
        specs += [full(wc1), full(bc1), full(wc2), full(bc2)]
    out_shapes = [jax.ShapeDtypeStruct((NK, 3), jnp.float32)] * 3 + [
        jax.ShapeDtypeStruct((NK, 1), jnp.float32)]
    out_specs = [row(3)] * 3 + [row(1)]
    return pl.pallas_call(
        _geom_body, grid=grid, in_specs=specs, out_specs=out_specs,
        out_shape=out_shapes)(*args)


def _wsum_body(win_ref, out_ref):
    out_ref[...] = jnp.sum(win_ref[...], axis=1, keepdims=True)


def _wsum_call(window2d):
    BN = 1000
    return pl.pallas_call(
        _wsum_body, grid=(N // BN,),
        in_specs=[pl.BlockSpec((BN, K), lambda i: (i, 0))],
        out_specs=pl.BlockSpec((BN, 1), lambda i: (i, 0)),
        out_shape=jax.ShapeDtypeStruct((N, 1), jnp.float32))(window2d)


def _linear_body(x_ref, w_ref, b_ref, o_ref):
    o_ref[...] = jnp.dot(x_ref[...], w_ref[...]) + b_ref[...]


def _linear_call(x, w, b):
    BN = 1000
    din, dout = w.shape
    return pl.pallas_call(
        _linear_body, grid=(N // BN,),
        in_specs=[pl.BlockSpec((BN, din), lambda i: (i, 0)),
                  pl.BlockSpec((din, dout), lambda i: (0, 0)),
                  pl.BlockSpec((1, dout), lambda i: (0, 0))],
        out_specs=pl.BlockSpec((BN, dout), lambda i: (i, 0)),
        out_shape=jax.ShapeDtypeStruct((N, dout), jnp.float32))(x, w, b)


def _mlp_body(h_ref, w1_ref, b1_ref, w2_ref, b2_ref, q_ref):
    t = _leaky(jnp.dot(h_ref[...], w1_ref[...]) + b1_ref[...])
    q = _leaky(jnp.dot(t, w2_ref[...]) + b2_ref[...])
    q_ref[...] = jnp.concatenate(
        [q, jnp.zeros_like(q)], axis=1)


def _mlp_call(h, w1, b1, w2, b2):
    BN = 1000
    din = h.shape[1]
    return pl.pallas_call(
        _mlp_body, grid=(N // BN,),
        in_specs=[pl.BlockSpec((BN, din), lambda i: (i, 0)),
                  pl.BlockSpec((din, H), lambda i: (0, 0)),
                  pl.BlockSpec((1, H), lambda i: (0, 0)),
                  pl.BlockSpec((H, H), lambda i: (0, 0)),
                  pl.BlockSpec((1, H), lambda i: (0, 0))],
        out_specs=pl.BlockSpec((BN, 2 * H), lambda i: (i, 0)),
        out_shape=jax.ShapeDtypeStruct((N, 2 * H), jnp.float32))(
            h, w1, b1, w2, b2)


def _post_body(theta, final, s_ref, ws_ref, res_ref, wm_ref, bm_ref, g_ref,
               b_ref, l1_ref, lb1_ref, l2_ref, lb2_ref, ch_ref,
               wo1_ref, bo1_ref, wo2_ref, bo2_ref, o_ref):
    F = jnp.dot(s_ref[...], wm_ref[...]) + ws_ref[...] * bm_ref[...]
    F1 = F * (g_ref[...] / math.sqrt(1.0 + EPS)) + b_ref[...]
    lin = jnp.maximum(jnp.dot(F1, l1_ref[...]) + lb1_ref[...], 0.0)
    lin = jnp.dot(lin, l2_ref[...]) + lb2_ref[...]
    sup = (1.0 - ALPHA) * lin + ALPHA * res_ref[...]
    out = theta * jnp.dot(sup, ch_ref[...]) + (1.0 - theta) * sup + lin
    h = jnp.maximum(out, 0.0)
    if final:
        o = _leaky(jnp.dot(h, wo1_ref[...]) + bo1_ref[...])
        o = _leaky(o * wo2_ref[...] + bo2_ref[...])
        o_ref[...] = jax.nn.sigmoid(o)
    else:
        o_ref[...] = h


def _post_call(theta, final, s, wsum, res, wm, bm, g, b, l1, lb1, l2, lb2,
               ch, wo1, bo1, wo2, bo2):
    BN = 1000
    full = lambda a: pl.BlockSpec(a.shape, lambda i: (0, 0))
    dout = 1 if final else H
    return pl.pallas_call(
        functools.partial(_post_body, theta, final), grid=(N // BN,),
        in_specs=[pl.BlockSpec((BN, 3 * H), lambda i: (i, 0)),
                  pl.BlockSpec((BN, 1), lambda i: (i, 0)),
                  pl.BlockSpec((BN, H), lambda i: (i, 0)),
                  full(wm), full(bm), full(g), full(b), full(l1), full(lb1),
                  full(l2), full(lb2), full(ch), full(wo1), full(bo1),
                  full(wo2), full(bo2)],
        out_specs=pl.BlockSpec((BN, dout), lambda i: (i, 0)),
        out_shape=jax.ShapeDtypeStruct((N, dout), jnp.float32))(
            s, wsum, res, wm, bm, g, b, l1, lb1, l2, lb2, ch, wo1, bo1,
            wo2, bo2)


def _sc_gather_body(q_hbm, idx_hbm, wgt_hbm, out_hbm, idx_all, w_all, rows,
                    out_half, *sems):
    wid = lax.axis_index("s") * NC + lax.axis_index("c")
    base_k = wid * (CHUNK * K)
    base_w = wid * (CHUNK * K * 3)
    base_o = wid * (CHUNK * 3 * H)
    HG = NG // 2
    GW = G * 3 * H

    pltpu.sync_copy(idx_hbm.at[pl.ds(base_k, CHUNK * K)], idx_all)
    pltpu.sync_copy(wgt_hbm.at[pl.ds(base_w, CHUNK * K * 3)], w_all)

    def gather(g, b):
        return pltpu.make_async_copy(
            q_hbm.at[idx_all.at[pl.ds(g * (G * K), G * K)]], rows.at[b],
            sems[b])

    for b in range(RING):
        gather(b, b).start()

    def compute(g, buf):
        off = (g % HG) * GW
        for i in range(G):
            node = g * G + i
            wbase = node * (K * 3)
            wv = [w_all[pl.ds(wbase + v * 16, 16)] for v in range(K * 3 // 16)]
            acc = [[jnp.zeros((16,), jnp.float32) for _ in range(H // 16)]
                   for _ in range(3)]
            for k in range(K):
                r = [buf[i * K + k, pl.ds(hc * 16, 16)]
                     for hc in range(H // 16)]
                for c in range(3):
                    p = 3 * k + c
                    lane = jnp.full((16,), p % 16, jnp.int32)
                    w_sp = wv[p // 16].at[lane].get(mode='promise_in_bounds')
                    for hc in range(H // 16):
                        acc[c][hc] = acc[c][hc] + w_sp * r[hc]
            for c in range(3):
                for hc in range(H // 16):
                    out_half[pl.ds(off + i * (3 * H) + c * H + hc * 16, 16)] = \
                        acc[c][hc]

    def body(t, carry):
        for b in range(RING):
            g = RING * t + b
            gather(g, b).wait()
            compute(g, rows.at[b])

            @pl.when(t + 1 < NG // RING)
            def _():
                gather(g + RING, b).start()

            @pl.when(g == HG - 1)
            def _():
                pltpu.sync_copy(out_half, out_hbm.at[pl.ds(base_o, HG * GW)])

            @pl.when(g == NG - 1)
            def _():
                pltpu.sync_copy(out_half,
                                out_hbm.at[pl.ds(base_o + HG * GW, HG * GW)])

        return carry

    lax.fori_loop(0, NG // RING, body, 0)


@functools.cache
def _get_sc_gather():
    return functools.partial(
        pl.kernel,
        out_type=jax.ShapeDtypeStruct((NP * 3 * H,), jnp.float32),
        mesh=plsc.VectorSubcoreMesh(core_axis_name="c", subcore_axis_name="s",
                                    num_cores=NC, num_subcores=NS),
        scratch_types=[
            pltpu.VMEM((CHUNK * K,), jnp.int32),
            pltpu.VMEM((CHUNK * K * 3,), jnp.float32),
            pltpu.VMEM((RING, G * K, 2 * H), jnp.float32),
            pltpu.VMEM(((NG // 2) * G * 3 * H,), jnp.float32),
        ] + [pltpu.SemaphoreType.DMA] * RING,
    )(_sc_gather_body)


def kernel(x, xyz_nb, xyz_id, dij, params):
    dij_flat = dij.reshape(NK, 3)
    xyz_flat = xyz_nb.reshape(NK, 3)
    convs = []
    for j in range(3):
        wc1, bc1, wc2, bc2 = params['conv%d' % j]
        convs.append((wc1, bc1.reshape(1, 16), wc2[:, :3],
                      bc2[:3].reshape(1, 3)))
    w0, w1, w2, window = _geom_call(dij_flat, xyz_flat, convs)
    wgt = [w0, w1, w2]
    wsum = _wsum_call(window.reshape(N, K))

    wfc, bfc = params['fc']
    residual = _linear_call(x, wfc, bfc.reshape(1, H))

    idx_pad = jnp.pad(xyz_id.astype(jnp.int32).reshape(-1), (0, NPK - NK))

    h = x
    o = None
    for j in range(3):
        w1_, b1_, w2_, b2_ = params['net_in%d' % j]
        q = _mlp_call(h, w1_, b1_.reshape(1, H), w2_, b2_.reshape(1, H))
        wgt_flat = jnp.pad(wgt[j].reshape(-1), (0, (NPK - NK) * 3))
        s = _get_sc_gather()(q, idx_pad, wgt_flat).reshape(NP, 3 * H)[:N]
        wm, bm = params['mlp']
        g, b = params['bn%d' % j]
        l1, lb1, l2, lb2 = params['lin%d' % j]
        wo1, bo1, wo2, bo2 = params['net_out']
        theta = min(1.0, math.log(LAMBDA / (j + 1) + 1.0))
        final = j == 2
        res = _post_call(theta, final, s, wsum, residual, wm,
                         bm.reshape(1, H), g.reshape(1, H), b.reshape(1, H),
                         l1, lb1.reshape(1, H), l2, lb2.reshape(1, H),
                         params['chen%d' % j], wo1, bo1.reshape(1, 1), wo2,
                         bo2.reshape(1, 1))
        if final:
            o = res
        else:
            h = res
    return o.reshape(N)


# speedup vs baseline: 3.6021x; 1.7187x over previous
import functools
import math

import jax
import jax.numpy as jnp
from jax import lax
from jax.experimental import pallas as pl
from jax.experimental.pallas import tpu as pltpu
from jax.experimental.pallas import tpu_sc as plsc

N = 10000
K = 32
D = 128
H = 64
NK = N * K
ROWS = NK // 128
THR = 13.0
EPS = 1e-5
ALPHA = 0.7
LAMBDA = 1.5

NC = 2
NS = 16
NW = NC * NS
CHUNK = 320
NL = NW * CHUNK
NPK = NL * K
G = 2
NG = CHUNK // G


def _leaky(v):
    return jnp.where(v >= 0, v, 0.2 * v)


def _geom_body(dx, dy, dz, xx, xy, xz,
               wc1_0, bc1_0, wc2_0, bc2_0, wc1_1, bc1_1, wc2_1, bc2_1,
               wc1_2, bc1_2, wc2_2, bc2_2,
               win_ref, w00, w01, w02, w10, w11, w12, w20, w21, w22):
    x0, x1, x2 = dx[...], dy[...], dz[...]
    d2 = x0 * x0 + x1 * x1 + x2 * x2
    win = jnp.exp(-d2 / (2.0 * THR * THR)) * (d2 < THR * THR).astype(jnp.float32)
    win_ref[...] = win
    a0, a1, a2 = xx[...], xy[...], xz[...]
    outs = [[w00, w01, w02], [w10, w11, w12], [w20, w21, w22]]
    for j, (wc1r, bc1r, wc2r, bc2r) in enumerate((
            (wc1_0, bc1_0, wc2_0, bc2_0),
            (wc1_1, bc1_1, wc2_1, bc2_1),
            (wc1_2, bc1_2, wc2_2, bc2_2))):
        w1 = wc1r[...]
        b1 = bc1r[...]
        w2 = wc2r[...]
        b2 = bc2r[...]
        p3 = [jnp.zeros(a0.shape, jnp.float32) for _ in range(3)]
        for i in range(16):
            t = jnp.maximum(
                a0 * w1[0, i] + a1 * w1[1, i] + a2 * w1[2, i] + b1[0, i], 0.0)
            for c in range(3):
                p3[c] = p3[c] + t * w2[i, c]
        for c in range(3):
            outs[j][c][...] = win * (p3[c] + b2[0, c])


def _geom_call(dij_flat, xyz_flat, convs):
    dp = dij_flat.T.reshape(3, ROWS, 128)
    xp = xyz_flat.T.reshape(3, ROWS, 128)
    plane = pl.BlockSpec((ROWS, 128), lambda: (0, 0))
    full = lambda a: pl.BlockSpec(a.shape, lambda: (0, 0))
    args = [dp[0], dp[1], dp[2], xp[0], xp[1], xp[2]]
    specs = [plane] * 6
    for (wc1, bc1, wc2, bc2) in convs:
        args += [wc1, bc1, wc2, bc2]
        specs += [full(wc1), full(bc1), full(wc2), full(bc2)]
    out_shapes = [jax.ShapeDtypeStruct((ROWS, 128), jnp.float32)] * 10
    out_specs = [plane] * 10
    return pl.pallas_call(
        _geom_body, in_specs=specs, out_specs=out_specs,
        out_shape=out_shapes)(*args)


def _wsum_body(win_ref, out_ref):
    out_ref[...] = jnp.sum(win_ref[...], axis=1, keepdims=True)


def _wsum_call(window2d):
    BN = 1000
    return pl.pallas_call(
        _wsum_body, grid=(N // BN,),
        in_specs=[pl.BlockSpec((BN, K), lambda i: (i, 0))],
        out_specs=pl.BlockSpec((BN, 1), lambda i: (i, 0)),
        out_shape=jax.ShapeDtypeStruct((N, 1), jnp.float32))(window2d)


BNC = 1024


def _mlp_t(x, w1t, b1, w2t, b2):
    t = _leaky(jnp.dot(w1t[...], x) + b1[...])
    return _leaky(jnp.dot(w2t[...], t) + b2[...])


def _pre_body(xt_ref, wfct, bfc, w1t, b1, w2t, b2, res_ref, q_ref):
    x = xt_ref[...]
    res_ref[...] = jnp.dot(wfct[...], x) + bfc[...]
    q_ref[...] = _mlp_t(x, w1t, b1, w2t, b2)


def _pre_call(xt, wfct, bfc, w1t, b1, w2t, b2):
    full = lambda a: pl.BlockSpec(a.shape, lambda i: (0, 0))
    return pl.pallas_call(
        _pre_body, grid=(NL // BNC,),
        in_specs=[pl.BlockSpec((D, BNC), lambda i: (0, i)),
                  full(wfct), full(bfc), full(w1t), full(b1), full(w2t),
                  full(b2)],
        out_specs=[pl.BlockSpec((H, BNC), lambda i: (0, i)),
                   pl.BlockSpec((H, BNC), lambda i: (0, i))],
        out_shape=[jax.ShapeDtypeStruct((H, NL), jnp.float32),
                   jax.ShapeDtypeStruct((H, NL), jnp.float32)])(
                       xt, wfct, bfc, w1t, b1, w2t, b2)


def _post_core(theta, s, ws, res, wmt, bm, g, b, l1t, lb1, l2t, lb2, cht):
    F = jnp.dot(wmt[...], s) + bm[...] * ws
    F1 = F * (g[...] / math.sqrt(1.0 + EPS)) + b[...]
    lin = jnp.maximum(jnp.dot(l1t[...], F1) + lb1[...], 0.0)
    lin = jnp.dot(l2t[...], lin) + lb2[...]
    sup = (1.0 - ALPHA) * lin + ALPHA * res
    out = theta * jnp.dot(cht[...], sup) + (1.0 - theta) * sup + lin
    return jnp.maximum(out, 0.0)


def _layer_body(theta, s_ref, ws_ref, res_ref, wmt, bm, g, b, l1t, lb1, l2t,
                lb2, cht, w1t, b1, w2t, b2, q_ref):
    h = _post_core(theta, s_ref[...], ws_ref[...], res_ref[...], wmt, bm, g,
                   b, l1t, lb1, l2t, lb2, cht)
    q_ref[...] = _mlp_t(h, w1t, b1, w2t, b2)


def _layer_call(theta, s, ws, res, wmt, bm, g, b, l1t, lb1, l2t, lb2, cht,
                w1t, b1, w2t, b2):
    full = lambda a: pl.BlockSpec(a.shape, lambda i: (0, 0))
    args = (s, ws, res, wmt, bm, g, b, l1t, lb1, l2t, lb2, cht, w1t, b1,
            w2t, b2)
    return pl.pallas_call(
        functools.partial(_layer_body, theta), grid=(NL // BNC,),
        in_specs=[pl.BlockSpec((3 * H, BNC), lambda i: (0, i)),
                  pl.BlockSpec((1, BNC), lambda i: (0, i)),
                  pl.BlockSpec((H, BNC), lambda i: (0, i))] +
                 [full(a) for a in args[3:]],
        out_specs=pl.BlockSpec((H, BNC), lambda i: (0, i)),
        out_shape=jax.ShapeDtypeStruct((H, NL), jnp.float32))(*args)


def _final_body(theta, s_ref, ws_ref, res_ref, wmt, bm, g, b, l1t, lb1, l2t,
                lb2, cht, wo1t, bo1, wo2, bo2, o_ref):
    h = _post_core(theta, s_ref[...], ws_ref[...], res_ref[...], wmt, bm, g,
                   b, l1t, lb1, l2t, lb2, cht)
    o = _leaky(jnp.dot(wo1t[...], h) + bo1[...])
    o = _leaky(o * wo2[...] + bo2[...])
    o_ref[...] = jax.nn.sigmoid(o)


def _final_call(theta, s, ws, res, wmt, bm, g, b, l1t, lb1, l2t, lb2, cht,
                wo1t, bo1, wo2, bo2):
    full = lambda a: pl.BlockSpec(a.shape, lambda i: (0, 0))
    args = (s, ws, res, wmt, bm, g, b, l1t, lb1, l2t, lb2, cht, wo1t, bo1,
            wo2, bo2)
    return pl.pallas_call(
        functools.partial(_final_body, theta), grid=(NL // BNC,),
        in_specs=[pl.BlockSpec((3 * H, BNC), lambda i: (0, i)),
                  pl.BlockSpec((1, BNC), lambda i: (0, i)),
                  pl.BlockSpec((H, BNC), lambda i: (0, i))] +
                 [full(a) for a in args[3:]],
        out_specs=pl.BlockSpec((1, BNC), lambda i: (0, i)),
        out_shape=jax.ShapeDtypeStruct((1, NL), jnp.float32))(*args)


def _sc_body(q_hbm, idx_hbm, wgt_hbm, out_hbm, qtab, idx_all, w_all, rows2,
             out_half, sem0, sem1):
    wid = lax.axis_index("s") * NC + lax.axis_index("c")
    base_k = wid * (CHUNK * K)
    base_w = wid * (CHUNK * K * 3)
    base_o = wid * (CHUNK * 3 * H)
    HG = NG // 2
    GW = G * 3 * H

    @pl.when(lax.axis_index("s") == 0)
    def _():
        pltpu.sync_copy(q_hbm, qtab)

    plsc.subcore_barrier()

    pltpu.sync_copy(idx_hbm.at[pl.ds(base_k, CHUNK * K)], idx_all)
    pltpu.sync_copy(wgt_hbm.at[pl.ds(base_w, CHUNK * K * 3)], w_all)

    def gather(g, buf, sem):
        return pltpu.make_async_copy(
            qtab.at[idx_all.at[pl.ds(g * (G * K), G * K)]], buf, sem)

    gather(0, rows2.at[0], sem0).start()
    gather(1, rows2.at[1], sem1).start()

    def compute(g, buf):
        off = (g % HG) * GW
        for i in range(G):
            node = g * G + i
            wbase = node * (K * 3)
            wv = [w_all[pl.ds(wbase + v * 16, 16)] for v in range(K * 3 // 16)]
            acc = [[jnp.zeros((16,), jnp.float32) for _ in range(H // 16)]
                   for _ in range(3)]
            for k in range(K):
                r = [buf[i * K + k, pl.ds(hc * 16, 16)]
                     for hc in range(H // 16)]
                for c in range(3):
                    p = 3 * k + c
                    lane = jnp.full((16,), p % 16, jnp.int32)
                    w_sp = wv[p // 16].at[lane].get(mode='promise_in_bounds')
                    for hc in range(H // 16):
                        acc[c][hc] = acc[c][hc] + w_sp * r[hc]
            for c in range(3):
                for hc in range(H // 16):
                    out_half[pl.ds(off + i * (3 * H) + c * H + hc * 16, 16)] = \
                        acc[c][hc]

    def body2(t, carry):
        g0 = 2 * t
        g1 = g0 + 1
        gather(g0, rows2.at[0], sem0).wait()
        compute(g0, rows2.at[0])

        @pl.when(t + 1 < NG // 2)
        def _():
            gather(g0 + 2, rows2.at[0], sem0).start()

        gather(g1, rows2.at[1], sem1).wait()
        compute(g1, rows2.at[1])

        @pl.when(t + 1 < NG // 2)
        def _():
            gather(g1 + 2, rows2.at[1], sem1).start()

        @pl.when(g1 == HG - 1)
        def _():
            pltpu.sync_copy(out_half, out_hbm.at[pl.ds(base_o, HG * GW)])

        @pl.when(g1 == NG - 1)
        def _():
            pltpu.sync_copy(out_half,
                            out_hbm.at[pl.ds(base_o + HG * GW, HG * GW)])

        return carry

    lax.fori_loop(0, NG // 2, body2, 0)


@functools.cache
def _get_sc():
    return functools.partial(
        pl.kernel,
        out_type=jax.ShapeDtypeStruct((NL * 3 * H,), jnp.float32),
        mesh=plsc.VectorSubcoreMesh(core_axis_name="c", subcore_axis_name="s",
                                    num_cores=NC, num_subcores=NS),
        scratch_types=[
            pltpu.VMEM_SHARED((N, H), jnp.float32),
            pltpu.VMEM((CHUNK * K,), jnp.int32),
            pltpu.VMEM((CHUNK * K * 3,), jnp.float32),
            pltpu.VMEM((2, G * K, H), jnp.float32),
            pltpu.VMEM(((NG // 2) * G * 3 * H,), jnp.float32),
            pltpu.SemaphoreType.DMA,
            pltpu.SemaphoreType.DMA,
        ],
    )(_sc_body)


def kernel(x, xyz_nb, xyz_id, dij, params):
    dij_flat = dij.reshape(NK, 3)
    xyz_flat = xyz_nb.reshape(NK, 3)
    convs = []
    for j in range(3):
        wc1, bc1, wc2, bc2 = params['conv%d' % j]
        convs.append((wc1, bc1.reshape(1, 16), wc2[:, :3],
                      bc2[:3].reshape(1, 3)))
    geo = _geom_call(dij_flat, xyz_flat, convs)
    window = geo[0].reshape(N, K)
    wsum = _wsum_call(window)
    wsumT = jnp.pad(wsum.reshape(1, N), ((0, 0), (0, NL - N)))

    wgt = []
    for j in range(3):
        w3 = jnp.stack([geo[1 + 3 * j + c].reshape(N, K) for c in range(3)],
                       axis=-1)
        wgt.append(jnp.pad(w3.reshape(NK * 3), (0, (NL - N) * K * 3)))

    idx_flat = jnp.pad(xyz_id.astype(jnp.int32).reshape(NK), (0, NPK - NK))

    xt = jnp.pad(x.T, ((0, 0), (0, NL - N)))
    wfc, bfc = params['fc']
    w1_, b1_, w2_, b2_ = params['net_in0']
    resT, qT = _pre_call(xt, wfc.T, bfc.reshape(H, 1), w1_.T,
                         b1_.reshape(H, 1), w2_.T, b2_.reshape(H, 1))

    wm, bm = params['mlp']
    wo1, bo1, wo2, bo2 = params['net_out']
    o = None
    for j in range(3):
        q_node = qT.T[:N]
        sflat = _get_sc()(q_node, idx_flat, wgt[j])
        sT = sflat.reshape(NL, 3 * H).T
        g, b = params['bn%d' % j]
        l1, lb1, l2, lb2 = params['lin%d' % j]
        theta = min(1.0, math.log(LAMBDA / (j + 1) + 1.0))
        common = (theta, sT, wsumT, resT, wm.T, bm.reshape(H, 1),
                  g.reshape(H, 1), b.reshape(H, 1), l1.T, lb1.reshape(H, 1),
                  l2.T, lb2.reshape(H, 1), params['chen%d' % j].T)
        if j < 2:
            nw1, nb1, nw2, nb2 = params['net_in%d' % (j + 1)]
            qT = _layer_call(*common, nw1.T, nb1.reshape(H, 1), nw2.T,
                             nb2.reshape(H, 1))
        else:
            o = _final_call(*common, wo1.T, bo1.reshape(1, 1), wo2,
                            bo2.reshape(1, 1))
    return o.reshape(NL)[:N]


# speedup vs baseline: 3.9854x; 1.1064x over previous
import functools
import math

import jax
import jax.numpy as jnp
from jax import lax
from jax.experimental import pallas as pl
from jax.experimental.pallas import tpu as pltpu
from jax.experimental.pallas import tpu_sc as plsc

N = 10000
K = 32
D = 128
H = 64
NK = N * K
ROWS = NK // 128
THR = 13.0
EPS = 1e-5
ALPHA = 0.7
LAMBDA = 1.5

NC = 2
NS = 16
NW = NC * NS
CHUNK = 320
NL = NW * CHUNK
NPK = NL * K
G = 2
NG = CHUNK // G


def _leaky(v):
    return jnp.where(v >= 0, v, 0.2 * v)


def _geom_body(dx, dy, dz, xx, xy, xz,
               wc1_0, bc1_0, wc2_0, bc2_0, wc1_1, bc1_1, wc2_1, bc2_1,
               wc1_2, bc1_2, wc2_2, bc2_2,
               win_ref, w00, w01, w02, w10, w11, w12, w20, w21, w22):
    x0, x1, x2 = dx[...], dy[...], dz[...]
    d2 = x0 * x0 + x1 * x1 + x2 * x2
    win = jnp.exp(-d2 / (2.0 * THR * THR)) * (d2 < THR * THR).astype(jnp.float32)
    win_ref[...] = win
    a0, a1, a2 = xx[...], xy[...], xz[...]
    outs = [[w00, w01, w02], [w10, w11, w12], [w20, w21, w22]]
    for j, (wc1r, bc1r, wc2r, bc2r) in enumerate((
            (wc1_0, bc1_0, wc2_0, bc2_0),
            (wc1_1, bc1_1, wc2_1, bc2_1),
            (wc1_2, bc1_2, wc2_2, bc2_2))):
        w1 = wc1r[...]
        b1 = bc1r[...]
        w2 = wc2r[...]
        b2 = bc2r[...]
        p3 = [jnp.zeros(a0.shape, jnp.float32) for _ in range(3)]
        for i in range(16):
            t = jnp.maximum(
                a0 * w1[0, i] + a1 * w1[1, i] + a2 * w1[2, i] + b1[0, i], 0.0)
            for c in range(3):
                p3[c] = p3[c] + t * w2[i, c]
        for c in range(3):
            outs[j][c][...] = win * (p3[c] + b2[0, c])


def _geom_call(dij_flat, xyz_flat, convs):
    dp = dij_flat.T.reshape(3, ROWS, 128)
    xp = xyz_flat.T.reshape(3, ROWS, 128)
    plane = pl.BlockSpec((ROWS, 128), lambda: (0, 0))
    full = lambda a: pl.BlockSpec(a.shape, lambda: (0, 0))
    args = [dp[0], dp[1], dp[2], xp[0], xp[1], xp[2]]
    specs = [plane] * 6
    for (wc1, bc1, wc2, bc2) in convs:
        args += [wc1, bc1, wc2, bc2]
        specs += [full(wc1), full(bc1), full(wc2), full(bc2)]
    out_shapes = [jax.ShapeDtypeStruct((ROWS, 128), jnp.float32)] * 10
    out_specs = [plane] * 10
    return pl.pallas_call(
        _geom_body, in_specs=specs, out_specs=out_specs,
        out_shape=out_shapes)(*args)


def _wsum_body(win_ref, out_ref):
    out_ref[...] = jnp.sum(win_ref[...], axis=1, keepdims=True)


def _wsum_call(window2d):
    BN = 1000
    return pl.pallas_call(
        _wsum_body, grid=(N // BN,),
        in_specs=[pl.BlockSpec((BN, K), lambda i: (i, 0))],
        out_specs=pl.BlockSpec((BN, 1), lambda i: (i, 0)),
        out_shape=jax.ShapeDtypeStruct((N, 1), jnp.float32))(window2d)


def _linear_body(x_ref, w_ref, b_ref, o_ref):
    o_ref[...] = jnp.dot(x_ref[...], w_ref[...]) + b_ref[...]


def _linear_call(x, w, b):
    BN = 1000
    din, dout = w.shape
    return pl.pallas_call(
        _linear_body, grid=(N // BN,),
        in_specs=[pl.BlockSpec((BN, din), lambda i: (i, 0)),
                  pl.BlockSpec((din, dout), lambda i: (0, 0)),
                  pl.BlockSpec((1, dout), lambda i: (0, 0))],
        out_specs=pl.BlockSpec((BN, dout), lambda i: (i, 0)),
        out_shape=jax.ShapeDtypeStruct((N, dout), jnp.float32))(x, w, b)


def _mlp_body(h_ref, w1_ref, b1_ref, w2_ref, b2_ref, q_ref):
    t = _leaky(jnp.dot(h_ref[...], w1_ref[...]) + b1_ref[...])
    q_ref[...] = _leaky(jnp.dot(t, w2_ref[...]) + b2_ref[...])


def _mlp_call(h, w1, b1, w2, b2):
    BN = 1000
    din = h.shape[1]
    return pl.pallas_call(
        _mlp_body, grid=(N // BN,),
        in_specs=[pl.BlockSpec((BN, din), lambda i: (i, 0)),
                  pl.BlockSpec((din, H), lambda i: (0, 0)),
                  pl.BlockSpec((1, H), lambda i: (0, 0)),
                  pl.BlockSpec((H, H), lambda i: (0, 0)),
                  pl.BlockSpec((1, H), lambda i: (0, 0))],
        out_specs=pl.BlockSpec((BN, H), lambda i: (i, 0)),
        out_shape=jax.ShapeDtypeStruct((N, H), jnp.float32))(h, w1, b1, w2, b2)


def _post_body(theta, final, s_ref, ws_ref, res_ref, wm_ref, bm_ref, g_ref,
               b_ref, l1_ref, lb1_ref, l2_ref, lb2_ref, ch_ref,
               wo1_ref, bo1_ref, wo2_ref, bo2_ref, o_ref):
    F = jnp.dot(s_ref[...], wm_ref[...]) + ws_ref[...] * bm_ref[...]
    F1 = F * (g_ref[...] / math.sqrt(1.0 + EPS)) + b_ref[...]
    lin = jnp.maximum(jnp.dot(F1, l1_ref[...]) + lb1_ref[...], 0.0)
    lin = jnp.dot(lin, l2_ref[...]) + lb2_ref[...]
    sup = (1.0 - ALPHA) * lin + ALPHA * res_ref[...]
    out = theta * jnp.dot(sup, ch_ref[...]) + (1.0 - theta) * sup + lin
    h = jnp.maximum(out, 0.0)
    if final:
        o = _leaky(jnp.dot(h, wo1_ref[...]) + bo1_ref[...])
        o = _leaky(o * wo2_ref[...] + bo2_ref[...])
        o_ref[...] = jax.nn.sigmoid(o)
    else:
        o_ref[...] = h


def _post_call(theta, final, s, wsum, res, wm, bm, g, b, l1, lb1, l2, lb2,
               ch, wo1, bo1, wo2, bo2):
    BN = 1000
    full = lambda a: pl.BlockSpec(a.shape, lambda i: (0, 0))
    dout = 1 if final else H
    return pl.pallas_call(
        functools.partial(_post_body, theta, final), grid=(N // BN,),
        in_specs=[pl.BlockSpec((BN, 3 * H), lambda i: (i, 0)),
                  pl.BlockSpec((BN, 1), lambda i: (i, 0)),
                  pl.BlockSpec((BN, H), lambda i: (i, 0)),
                  full(wm), full(bm), full(g), full(b), full(l1), full(lb1),
                  full(l2), full(lb2), full(ch), full(wo1), full(bo1),
                  full(wo2), full(bo2)],
        out_specs=pl.BlockSpec((BN, dout), lambda i: (i, 0)),
        out_shape=jax.ShapeDtypeStruct((N, dout), jnp.float32))(
            s, wsum, res, wm, bm, g, b, l1, lb1, l2, lb2, ch, wo1, bo1,
            wo2, bo2)


def _sc_body(q_hbm, idx_hbm, wgt_hbm, out_hbm, qtab, idx_all, w_all, rows2,
             out_half, sem0, sem1):
    wid = lax.axis_index("s") * NC + lax.axis_index("c")
    base_k = wid * (CHUNK * K)
    base_w = wid * (CHUNK * K * 3)
    base_o = wid * (CHUNK * 3 * H)
    HG = NG // 2
    GW = G * 3 * H

    @pl.when(lax.axis_index("s") == 0)
    def _():
        pltpu.sync_copy(q_hbm, qtab)

    plsc.subcore_barrier()

    pltpu.sync_copy(idx_hbm.at[pl.ds(base_k, CHUNK * K)], idx_all)
    pltpu.sync_copy(wgt_hbm.at[pl.ds(base_w, CHUNK * K * 3)], w_all)

    def gather(g, buf, sem):
        return pltpu.make_async_copy(
            qtab.at[idx_all.at[pl.ds(g * (G * K), G * K)]], buf, sem)

    gather(0, rows2.at[0], sem0).start()
    gather(1, rows2.at[1], sem1).start()

    def compute(g, buf):
        off = (g % HG) * GW
        for i in range(G):
            node = g * G + i
            wbase = node * (K * 3)
            wv = [w_all[pl.ds(wbase + v * 16, 16)] for v in range(K * 3 // 16)]
            acc = [[jnp.zeros((16,), jnp.float32) for _ in range(H // 16)]
                   for _ in range(3)]
            for k in range(K):
                r = [buf[i * K + k, pl.ds(hc * 16, 16)]
                     for hc in range(H // 16)]
                for c in range(3):
                    p = 3 * k + c
                    lane = jnp.full((16,), p % 16, jnp.int32)
                    w_sp = wv[p // 16].at[lane].get(mode='promise_in_bounds')
                    for hc in range(H // 16):
                        acc[c][hc] = acc[c][hc] + w_sp * r[hc]
            for c in range(3):
                for hc in range(H // 16):
                    out_half[pl.ds(off + i * (3 * H) + c * H + hc * 16, 16)] = \
                        acc[c][hc]

    def body2(t, carry):
        g0 = 2 * t
        g1 = g0 + 1
        gather(g0, rows2.at[0], sem0).wait()
        compute(g0, rows2.at[0])

        @pl.when(t + 1 < NG // 2)
        def _():
            gather(g0 + 2, rows2.at[0], sem0).start()

        gather(g1, rows2.at[1], sem1).wait()
        compute(g1, rows2.at[1])

        @pl.when(t + 1 < NG // 2)
        def _():
            gather(g1 + 2, rows2.at[1], sem1).start()

        @pl.when(g1 == HG - 1)
        def _():
            pltpu.sync_copy(out_half, out_hbm.at[pl.ds(base_o, HG * GW)])

        @pl.when(g1 == NG - 1)
        def _():
            pltpu.sync_copy(out_half,
                            out_hbm.at[pl.ds(base_o + HG * GW, HG * GW)])

        return carry

    lax.fori_loop(0, NG // 2, body2, 0)


@functools.cache
def _get_sc():
    return functools.partial(
        pl.kernel,
        out_type=jax.ShapeDtypeStruct((NL * 3 * H,), jnp.float32),
        mesh=plsc.VectorSubcoreMesh(core_axis_name="c", subcore_axis_name="s",
                                    num_cores=NC, num_subcores=NS),
        scratch_types=[
            pltpu.VMEM_SHARED((N, H), jnp.float32),
            pltpu.VMEM((CHUNK * K,), jnp.int32),
            pltpu.VMEM((CHUNK * K * 3,), jnp.float32),
            pltpu.VMEM((2, G * K, H), jnp.float32),
            pltpu.VMEM(((NG // 2) * G * 3 * H,), jnp.float32),
            pltpu.SemaphoreType.DMA,
            pltpu.SemaphoreType.DMA,
        ],
    )(_sc_body)


def kernel(x, xyz_nb, xyz_id, dij, params):
    dij_flat = dij.reshape(NK, 3)
    xyz_flat = xyz_nb.reshape(NK, 3)
    convs = []
    for j in range(3):
        wc1, bc1, wc2, bc2 = params['conv%d' % j]
        convs.append((wc1, bc1.reshape(1, 16), wc2[:, :3],
                      bc2[:3].reshape(1, 3)))
    geo = _geom_call(dij_flat, xyz_flat, convs)
    window = geo[0].reshape(N, K)
    wsum = _wsum_call(window)

    wgt = []
    for j in range(3):
        w3 = jnp.stack([geo[1 + 3 * j + c].reshape(N, K) for c in range(3)],
                       axis=-1)
        wgt.append(jnp.pad(w3.reshape(NK * 3), (0, (NL - N) * K * 3)))

    idx_flat = jnp.pad(xyz_id.astype(jnp.int32).reshape(NK), (0, NPK - NK))

    wfc, bfc = params['fc']
    residual = _linear_call(x, wfc, bfc.reshape(1, H))

    wm, bm = params['mlp']
    wo1, bo1, wo2, bo2 = params['net_out']
    h = x
    o = None
    for j in range(3):
        w1_, b1_, w2_, b2_ = params['net_in%d' % j]
        q = _mlp_call(h, w1_, b1_.reshape(1, H), w2_, b2_.reshape(1, H))
        sflat = _get_sc()(q, idx_flat, wgt[j])
        s = sflat.reshape(NL, 3 * H)[:N]
        g, b = params['bn%d' % j]
        l1, lb1, l2, lb2 = params['lin%d' % j]
        theta = min(1.0, math.log(LAMBDA / (j + 1) + 1.0))
        final = j == 2
        res = _post_call(theta, final, s, wsum, residual, wm,
                         bm.reshape(1, H), g.reshape(1, H), b.reshape(1, H),
                         l1, lb1.reshape(1, H), l2, lb2.reshape(1, H),
                         params['chen%d' % j], wo1, bo1.reshape(1, 1), wo2,
                         bo2.reshape(1, 1))
        if final:
            o = res
        else:
            h = res
    return o.reshape(N)
